# Initial kernel scaffold; baseline (speedup 1.0000x reference)
import jax
import jax.numpy as jnp
from jax.experimental import pallas as pl


def kernel(x, batch, batch_size, d_rows, d_cols, d_vals, d_index, Wq, Wk, Wv):
    raise NotImplementedError("write your pallas kernel here")


# speedup vs baseline: 72.0715x; 72.0715x over previous
import functools
import jax
import jax.numpy as jnp
from jax import lax
from jax.experimental import pallas as pl
from jax.experimental.pallas import tpu as pltpu
from jax.experimental.pallas import tpu_sc as plsc
from math import sqrt

B = 16
N_PER = 4096
M = 3 * N_PER
D = 256
H = 8
DH = D // H
NNZ = 196608
NORM = 1.0 / sqrt(DH)

NC = 2
NS = 16
NW = NC * NS
LANE = 128
ROWS_PER_W = NNZ // 2 // LANE
CHUNK_ROWS = 64
N_CHUNKS = ROWS_PER_W // CHUNK_ROWS
W_WORDS = 8 * N_PER


def _sc_weights(d_rows, d_cols, d_vals, d_index):
    rows_r = d_rows.reshape(B, NNZ // LANE, LANE)
    cols_r = d_cols.reshape(B, NNZ // LANE, LANE)
    vals_r = d_vals.reshape(B, NNZ // LANE, LANE)
    mesh = plsc.VectorSubcoreMesh(core_axis_name="c", subcore_axis_name="s")

    @functools.partial(
        pl.kernel,
        mesh=mesh,
        compiler_params=pltpu.CompilerParams(needs_layout_passes=False),
        out_type=jax.ShapeDtypeStruct((NW, W_WORDS), jnp.float32),
        scratch_types=[
            pltpu.VMEM((M,), jnp.int32),
            pltpu.VMEM((CHUNK_ROWS, LANE), jnp.int32),
            pltpu.VMEM((CHUNK_ROWS, LANE), jnp.int32),
            pltpu.VMEM((CHUNK_ROWS, LANE), jnp.float32),
            pltpu.VMEM((CHUNK_ROWS, LANE), jnp.int32),
            pltpu.VMEM((W_WORDS,), jnp.float32),
            pltpu.VMEM_SHARED((NS * W_WORDS,), jnp.float32),
            pltpu.SemaphoreType.DMA,
        ],
    )
    def k(rows_hbm, cols_hbm, vals_hbm, dindex_hbm, out_hbm,
          dindex_v, rows_v, cols_v, vals_v, idx_v, wz_v, w_sh, sem):
        cid = lax.axis_index("c")
        sid = lax.axis_index("s")
        wid = sid * NC + cid
        g = sid
        half = cid
        wbase = sid * W_WORDS

        def zbody(i, carry):
            wz_v[pl.ds(i * 16, 16)] = jnp.zeros((16,), jnp.float32)
            return carry
        lax.fori_loop(0, W_WORDS // 16, zbody, 0)
        pltpu.sync_copy(wz_v, w_sh.at[pl.ds(wbase, W_WORDS)])

        pltpu.sync_copy(dindex_hbm.at[g], dindex_v)

        def chunk_body(c, carry):
            base = half * ROWS_PER_W + c * CHUNK_ROWS
            pltpu.sync_copy(rows_hbm.at[g, pl.ds(base, CHUNK_ROWS)], rows_v)
            pltpu.sync_copy(cols_hbm.at[g, pl.ds(base, CHUNK_ROWS)], cols_v)
            pltpu.sync_copy(vals_hbm.at[g, pl.ds(base, CHUNK_ROWS)], vals_v)

            def compute_row(j, carry2):
                for t in range(LANE // 16):
                    rows = rows_v[j, pl.ds(t * 16, 16)]
                    cols = cols_v[j, pl.ds(t * 16, 16)]
                    s = plsc.load_gather(dindex_v, [rows])
                    idx_v[j, pl.ds(t * 16, 16)] = wbase + s * N_PER + cols
                return carry2
            lax.fori_loop(0, CHUNK_ROWS, compute_row, 0)

            def fire(j, carry2):
                pltpu.async_copy(vals_v.at[j], w_sh.at[idx_v.at[j]], sem,
                                 add=True)
                return carry2
            lax.fori_loop(0, CHUNK_ROWS, fire, 0)

            def drain(j, carry2):
                pltpu.make_async_copy(
                    vals_v.at[j], w_sh.at[idx_v.at[j]], sem).wait()
                return carry2
            lax.fori_loop(0, CHUNK_ROWS, drain, 0)
            return carry
        lax.fori_loop(0, N_CHUNKS, chunk_body, 0)

        pltpu.sync_copy(w_sh.at[pl.ds(wbase, W_WORDS)], out_hbm.at[wid])

    return k(rows_r, cols_r, vals_r, d_index)


def _tc_body(x_ref, wp_ref, wq_ref, wk_ref, wv_ref, o_ref):
    w = wp_ref[0] + wp_ref[1]
    xd = jnp.dot(w, x_ref[...], preferred_element_type=jnp.float32,
                 precision=lax.Precision.HIGHEST)

    wq = wq_ref[...]
    wk = wk_ref[...]
    wv = wv_ref[...]
    tdot = lambda a, b: lax.dot_general(
        a, b, (((1,), (1,)), ((), ())), preferred_element_type=jnp.float32)
    q = tdot(xd, wq)
    k = tdot(xd, wk)
    v = tdot(xd, wv)

    col = lax.broadcasted_iota(jnp.int32, (8, D), 1) // DH
    tok = lax.broadcasted_iota(jnp.int32, (8, 8), 1)
    acc = jnp.zeros((8, D), jnp.float32)
    for h in range(H):
        hmask = (col == h)
        qh = jnp.where(hmask, q, 0.0)
        dist = tdot(qh, k) * NORM
        dist = jnp.where(tok < 3, dist, -1e30)
        dist = jax.nn.softmax(dist, axis=-1)
        vh = jnp.where(hmask, v, 0.0)
        acc = acc + jnp.dot(dist, vh, preferred_element_type=jnp.float32)
    o_ref[0] = acc


def _tc_attention(x, wp, wq, wk, wv):
    return pl.pallas_call(
        _tc_body,
        grid=(B,),
        in_specs=[
            pl.BlockSpec((N_PER, D), lambda g: (g, 0)),
            pl.BlockSpec((2, 8, N_PER), lambda g: (g, 0, 0)),
            pl.BlockSpec((D, D), lambda g: (0, 0)),
            pl.BlockSpec((D, D), lambda g: (0, 0)),
            pl.BlockSpec((D, D), lambda g: (0, 0)),
        ],
        out_specs=pl.BlockSpec((1, 8, D), lambda g: (g, 0, 0)),
        out_shape=jax.ShapeDtypeStruct((B, 8, D), jnp.float32),
    )(x, wp, wq, wk, wv)


def kernel(x, batch, batch_size, d_rows, d_cols, d_vals, d_index, Wq, Wk, Wv):
    wp = _sc_weights(d_rows, d_cols, d_vals, d_index)
    wp = wp.reshape(NW, 8, N_PER)
    out8 = _tc_attention(x, wp, Wq, Wk, Wv)
    return out8[:, :3, :].reshape(B, 3 * D)


# speedup vs baseline: 94.3310x; 1.3089x over previous
import functools
import jax
import jax.numpy as jnp
from jax import lax
from jax.experimental import pallas as pl
from jax.experimental.pallas import tpu as pltpu
from jax.experimental.pallas import tpu_sc as plsc
from math import sqrt

B = 16
N_PER = 4096
M = 3 * N_PER
D = 256
H = 8
DH = D // H
NNZ = 196608
NORM = 1.0 / sqrt(DH)

NC = 2
NS = 16
NW = NC * NS
LANE = 128
E_PER_W = NNZ // 2
CHUNK = 8192
CROWS = CHUNK // LANE
N_CHUNKS = E_PER_W // CHUNK
W_WORDS = 3 * N_PER


def _sc_weights(d_rows, d_cols, d_vals, d_index):
    mesh = plsc.VectorSubcoreMesh(core_axis_name="c", subcore_axis_name="s")

    @functools.partial(
        pl.kernel,
        mesh=mesh,
        compiler_params=pltpu.CompilerParams(needs_layout_passes=False),
        out_type=jax.ShapeDtypeStruct((NW, W_WORDS), jnp.float32),
        scratch_types=[
            pltpu.VMEM((M,), jnp.int32),
            pltpu.VMEM((2, CHUNK), jnp.int32),
            pltpu.VMEM((2, CHUNK), jnp.int32),
            pltpu.VMEM((2, CHUNK), jnp.float32),
            pltpu.VMEM((2 * CROWS, LANE), jnp.int32),
            pltpu.VMEM((W_WORDS,), jnp.float32),
            pltpu.VMEM_SHARED((NS * W_WORDS,), jnp.float32),
            pltpu.SemaphoreType.DMA,
            pltpu.SemaphoreType.DMA,
        ],
    )
    def k(rows_hbm, cols_hbm, vals_hbm, dindex_hbm, out_hbm,
          dindex_v, rows_v, cols_v, vals_v, idx_v, wz_v, w_sh, sem_in, sem_sc):
        cid = lax.axis_index("c")
        sid = lax.axis_index("s")
        wid = sid * NC + cid
        g = sid
        ebase = cid * E_PER_W
        wbase = sid * W_WORDS

        def zbody(i, carry):
            wz_v[pl.ds(i * 16, 16)] = jnp.zeros((16,), jnp.float32)
            return carry
        lax.fori_loop(0, W_WORDS // 16, zbody, 0)
        pltpu.sync_copy(wz_v, w_sh.at[pl.ds(wbase, W_WORDS)])

        def start_in(c, b):
            off = ebase + c * CHUNK
            pltpu.async_copy(rows_hbm.at[g, pl.ds(off, CHUNK)],
                             rows_v.at[b], sem_in)
            pltpu.async_copy(cols_hbm.at[g, pl.ds(off, CHUNK)],
                             cols_v.at[b], sem_in)
            pltpu.async_copy(vals_hbm.at[g, pl.ds(off, CHUNK)],
                             vals_v.at[b], sem_in)

        def wait_in(c, b):
            off = ebase + c * CHUNK
            pltpu.make_async_copy(rows_hbm.at[g, pl.ds(off, CHUNK)],
                                  rows_v.at[b], sem_in).wait()
            pltpu.make_async_copy(cols_hbm.at[g, pl.ds(off, CHUNK)],
                                  cols_v.at[b], sem_in).wait()
            pltpu.make_async_copy(vals_hbm.at[g, pl.ds(off, CHUNK)],
                                  vals_v.at[b], sem_in).wait()

        def fire_scatter(b):
            def fire(j, carry):
                pltpu.async_copy(vals_v.at[b, pl.ds(j * LANE, LANE)],
                                 w_sh.at[idx_v.at[b * CROWS + j]],
                                 sem_sc, add=True)
                return carry
            lax.fori_loop(0, CROWS, fire, 0)

        def drain_scatter(b):
            def drain(j, carry):
                pltpu.make_async_copy(vals_v.at[b, pl.ds(j * LANE, LANE)],
                                      w_sh.at[idx_v.at[b * CROWS + j]],
                                      sem_sc).wait()
                return carry
            lax.fori_loop(0, CROWS, drain, 0)

        start_in(0, 0)
        pltpu.sync_copy(dindex_hbm.at[g], dindex_v)

        def chunk_body(c, carry):
            b = lax.rem(c, 2)
            wait_in(c, b)

            def compute_row(j, carry2):
                for t in range(LANE // 16):
                    rows = rows_v[b, pl.ds(j * LANE + t * 16, 16)]
                    cols = cols_v[b, pl.ds(j * LANE + t * 16, 16)]
                    s = plsc.load_gather(dindex_v, [rows])
                    idx_v[b * CROWS + j, pl.ds(t * 16, 16)] = \
                        wbase + s * N_PER + cols
                return carry2
            lax.fori_loop(0, CROWS, compute_row, 0)

            @pl.when(c > 0)
            def _():
                drain_scatter(1 - b)

            @pl.when(c + 1 < N_CHUNKS)
            def _():
                start_in(c + 1, 1 - b)

            fire_scatter(b)
            return carry
        lax.fori_loop(0, N_CHUNKS, chunk_body, 0)

        drain_scatter(lax.rem(N_CHUNKS - 1, 2))
        pltpu.sync_copy(w_sh.at[pl.ds(wbase, W_WORDS)], out_hbm.at[wid])

    return k(d_rows, d_cols, d_vals, d_index)


def _tc_body(x_ref, wp_ref, wq_ref, wk_ref, wv_ref, o_ref):
    w = wp_ref[0] + wp_ref[1]
    xd = jnp.dot(w, x_ref[...], preferred_element_type=jnp.float32,
                 precision=lax.Precision.HIGHEST)

    wq = wq_ref[...]
    wk = wk_ref[...]
    wv = wv_ref[...]
    tdot = lambda a, b: lax.dot_general(
        a, b, (((1,), (1,)), ((), ())), preferred_element_type=jnp.float32)
    q = tdot(xd, wq)
    k = tdot(xd, wk)
    v = tdot(xd, wv)

    col = lax.broadcasted_iota(jnp.int32, (3, D), 1) // DH
    acc = jnp.zeros((3, D), jnp.float32)
    for h in range(H):
        hmask = (col == h)
        qh = jnp.where(hmask, q, 0.0)
        dist = tdot(qh, k) * NORM
        dist = jax.nn.softmax(dist, axis=-1)
        vh = jnp.where(hmask, v, 0.0)
        acc = acc + jnp.dot(dist, vh, preferred_element_type=jnp.float32)
    o_ref[0] = acc


def _tc_attention(x, wp, wq, wk, wv):
    return pl.pallas_call(
        _tc_body,
        grid=(B,),
        in_specs=[
            pl.BlockSpec((N_PER, D), lambda g: (g, 0)),
            pl.BlockSpec((2, 3, N_PER), lambda g: (g, 0, 0)),
            pl.BlockSpec((D, D), lambda g: (0, 0)),
            pl.BlockSpec((D, D), lambda g: (0, 0)),
            pl.BlockSpec((D, D), lambda g: (0, 0)),
        ],
        out_specs=pl.BlockSpec((1, 3, D), lambda g: (g, 0, 0)),
        out_shape=jax.ShapeDtypeStruct((B, 3, D), jnp.float32),
    )(x, wp, wq, wk, wv)


def kernel(x, batch, batch_size, d_rows, d_cols, d_vals, d_index, Wq, Wk, Wv):
    wp = _sc_weights(d_rows, d_cols, d_vals, d_index)
    wp = wp.reshape(NW, 3, N_PER)
    out3 = _tc_attention(x, wp, Wq, Wk, Wv)
    return out3.reshape(B, 3 * D)
